# baseline (speedup 1.0000x reference)
import functools

import jax
import jax.numpy as jnp
from jax.experimental import compute_on
from jax.experimental import pallas as pl
from jax.experimental.pallas import tpu as pltpu
from jax.experimental.pallas import tpu_sc as plsc

N = 1000000

TC_N = 786432
ROWS, COLS = 768, 1024
BLOCK_ROWS = 256
GRID = ROWS // BLOCK_ROWS
STRIP = 8

SC_CHUNK = 6688
SC_LAST = N - SC_CHUNK
NW = 32

_KS0 = 0
_KS1 = 42
_KS2 = _KS0 ^ _KS1 ^ 0x1BD11BDA
_ROT = ((13, 15, 26, 6), (17, 29, 16, 24))


def _rotl(x, d):
    return (x << jnp.uint32(d)) | (x >> jnp.uint32(32 - d))


def _threefry_bits(c2):
    ks = (jnp.uint32(_KS0), jnp.uint32(_KS1), jnp.uint32(_KS2))
    x0 = jnp.full(c2.shape, _KS0, jnp.uint32)
    x1 = c2 + ks[1]

    def rounds(x0, x1, rs):
        for r in rs:
            x0 = x0 + x1
            x1 = _rotl(x1, r)
            x1 = x0 ^ x1
        return x0, x1

    x0, x1 = rounds(x0, x1, _ROT[0])
    x0, x1 = x0 + ks[1], x1 + ks[2] + jnp.uint32(1)
    x0, x1 = rounds(x0, x1, _ROT[1])
    x0, x1 = x0 + ks[2], x1 + ks[0] + jnp.uint32(2)
    x0, x1 = rounds(x0, x1, _ROT[0])
    x0, x1 = x0 + ks[0], x1 + ks[1] + jnp.uint32(3)
    x0, x1 = rounds(x0, x1, _ROT[1])
    x0, x1 = x0 + ks[1], x1 + ks[2] + jnp.uint32(4)
    x0, x1 = rounds(x0, x1, _ROT[0])
    x0, x1 = x0 + ks[2], x1 + ks[0] + jnp.uint32(5)
    return x0 ^ x1


_TINY = 1.1754943508222875e-38


def _uniform_from_bits(bits):
    fbits = (bits >> jnp.uint32(9)) | jnp.uint32(0x3F800000)
    f = jax.lax.bitcast_convert_type(fbits, jnp.float32) - jnp.float32(1.0)
    return jnp.maximum(f, jnp.float32(_TINY))


def _tc_body(w_ref, idx_out, val_out, m_acc, idx_acc):
    j = pl.program_id(0)

    row = jax.lax.broadcasted_iota(jnp.int32, (STRIP, COLS), 0)
    col = jax.lax.broadcasted_iota(jnp.int32, (STRIP, COLS), 1)
    pos0 = j * BLOCK_ROWS * COLS + row * COLS + col

    def step(i, carry):
        m_vec, idx_vec = carry
        pos = pos0 + i * (STRIP * COLS)
        w = w_ref[pl.ds(i * STRIP, STRIP), :]
        u = _uniform_from_bits(_threefry_bits(pos.astype(jnp.uint32)))
        z = -jnp.log(-jnp.log(u)) + jnp.log(w)
        upd = z > m_vec
        m_vec = jnp.where(upd, z, m_vec)
        idx_vec = jnp.where(upd, pos, idx_vec)
        return m_vec, idx_vec

    m0 = jnp.where(j == 0, jnp.full((STRIP, COLS), -jnp.inf, jnp.float32),
                   m_acc[...])
    i0 = jnp.where(j == 0, jnp.full((STRIP, COLS), 2**31 - 1, jnp.int32),
                   idx_acc[...])
    m_vec, idx_vec = jax.lax.fori_loop(
        0, BLOCK_ROWS // STRIP, step, (m0, i0), unroll=4)
    m_acc[...] = m_vec
    idx_acc[...] = idx_vec

    @pl.when(j == GRID - 1)
    def _():
        m = jnp.max(m_vec)
        idx_out[0] = jnp.min(
            jnp.where(m_vec == m, idx_vec, jnp.int32(2**31 - 1)))
        val_out[0] = m


def _tc_call(wp):
    return pl.pallas_call(
        _tc_body,
        grid=(GRID,),
        in_specs=[pl.BlockSpec((BLOCK_ROWS, COLS), lambda j: (j, 0))],
        out_specs=[pl.BlockSpec(memory_space=pltpu.SMEM),
                   pl.BlockSpec(memory_space=pltpu.SMEM)],
        out_shape=[jax.ShapeDtypeStruct((1,), jnp.int32),
                   jax.ShapeDtypeStruct((1,), jnp.float32)],
        scratch_shapes=[
            pltpu.VMEM((STRIP, COLS), jnp.float32),
            pltpu.VMEM((STRIP, COLS), jnp.int32),
        ],
    )(wp)


_SQRT2 = 1.4142135623730951
_LN2 = 0.6931471805599453


def _sc_body(w_hbm, s_out, i_out, w_v, res_s, res_i):
    c = jax.lax.axis_index("c")
    s_ax = jax.lax.axis_index("s")
    wid = s_ax * 2 + c
    base = pl.multiple_of(jnp.minimum(TC_N + wid * SC_CHUNK, SC_LAST), 16)
    pltpu.sync_copy(w_hbm.at[pl.ds(base, SC_CHUNK)], w_v)
    lane = jax.lax.iota(jnp.int32, 16)

    def step(i, carry):
        bs, bi = carry
        w = w_v[pl.ds(i * 16, 16)]
        pos = base + i * 16 + lane
        u = _uniform_from_bits(_threefry_bits(pos.astype(jnp.uint32)))
        ub = jax.lax.bitcast_convert_type(u, jnp.uint32)
        k = (ub >> jnp.uint32(23)).astype(jnp.int32) - 127
        m = jax.lax.bitcast_convert_type(
            (ub & jnp.uint32(0x007FFFFF)) | jnp.uint32(0x3F800000),
            jnp.float32)
        big = m > jnp.float32(_SQRT2)
        m = jnp.where(big, m * jnp.float32(0.5), m)
        k = jnp.where(big, k + 1, k)
        t = (m - jnp.float32(1.0)) / (m + jnp.float32(1.0))
        t2 = t * t
        p = t * (jnp.float32(1.0) + t2 * (jnp.float32(1 / 3) + t2 * (
            jnp.float32(1 / 5) + t2 * (jnp.float32(1 / 7)
                                       + t2 * jnp.float32(1 / 9)))))
        lnu = k.astype(jnp.float32) * jnp.float32(_LN2) + jnp.float32(2.0) * p
        s = w / (-lnu)
        upd = s > bs
        bs = jnp.where(upd, s, bs)
        bi = jnp.where(upd, pos, bi)
        return bs, bi

    bs0 = jnp.full((16,), -1.0, jnp.float32)
    bi0 = jnp.full((16,), 2**31 - 1, jnp.int32)
    bs, bi = jax.lax.fori_loop(0, SC_CHUNK // 16, step, (bs0, bi0))
    res_s[...] = bs
    res_i[...] = bi
    pltpu.sync_copy(res_s, s_out.at[wid])
    pltpu.sync_copy(res_i, i_out.at[wid])


_sc_call = functools.partial(
    pl.kernel,
    out_type=[jax.ShapeDtypeStruct((NW, 16), jnp.float32),
              jax.ShapeDtypeStruct((NW, 16), jnp.int32)],
    mesh=plsc.VectorSubcoreMesh(core_axis_name="c", subcore_axis_name="s"),
    scratch_types=[
        pltpu.VMEM((SC_CHUNK,), jnp.float32),
        pltpu.VMEM((16,), jnp.float32),
        pltpu.VMEM((16,), jnp.int32),
    ],
)(_sc_body)


def kernel(weights):
    with compute_on.compute_on("tpu_sparsecore"):
        sc_s, sc_i = _sc_call(weights)
    wp = weights[:TC_N].reshape(ROWS, COLS)
    tc_idx, tc_val = _tc_call(wp)

    m = jnp.max(sc_s)
    sc_idx = jnp.min(jnp.where(sc_s == m, sc_i, jnp.int32(2**31 - 1)))
    z_sc = jnp.log(m)
    idx = jnp.where(z_sc > tc_val[0], sc_idx, tc_idx[0])
    return idx[None]


# speedup vs baseline: 1.8634x; 1.8634x over previous
import jax
import jax.numpy as jnp
from jax.experimental import pallas as pl
from jax.experimental.pallas import tpu as pltpu

N = 1000000
ROWS, COLS = 1024, 1024
PAD = ROWS * COLS
BLOCK_ROWS = 256
GRID = ROWS // BLOCK_ROWS
STRIP = 8

_KS0 = 0
_KS1 = 42
_KS2 = _KS0 ^ _KS1 ^ 0x1BD11BDA
_ROT = ((13, 15, 26, 6), (17, 29, 16, 24))


def _rotl(x, d):
    return (x << jnp.uint32(d)) | (x >> jnp.uint32(32 - d))


def _threefry_bits(c2):
    ks = (jnp.uint32(_KS0), jnp.uint32(_KS1), jnp.uint32(_KS2))
    x0 = jnp.full(c2.shape, _KS0, jnp.uint32)
    x1 = c2 + ks[1]

    def rounds(x0, x1, rs):
        for r in rs:
            x0 = x0 + x1
            x1 = _rotl(x1, r)
            x1 = x0 ^ x1
        return x0, x1

    x0, x1 = rounds(x0, x1, _ROT[0])
    x0, x1 = x0 + ks[1], x1 + ks[2] + jnp.uint32(1)
    x0, x1 = rounds(x0, x1, _ROT[1])
    x0, x1 = x0 + ks[2], x1 + ks[0] + jnp.uint32(2)
    x0, x1 = rounds(x0, x1, _ROT[0])
    x0, x1 = x0 + ks[0], x1 + ks[1] + jnp.uint32(3)
    x0, x1 = rounds(x0, x1, _ROT[1])
    x0, x1 = x0 + ks[1], x1 + ks[2] + jnp.uint32(4)
    x0, x1 = rounds(x0, x1, _ROT[0])
    x0, x1 = x0 + ks[2], x1 + ks[0] + jnp.uint32(5)
    return x0 ^ x1


def _body(w_ref, out_ref, m_acc, idx_acc):
    j = pl.program_id(0)

    row = jax.lax.broadcasted_iota(jnp.int32, (STRIP, COLS), 0)
    col = jax.lax.broadcasted_iota(jnp.int32, (STRIP, COLS), 1)
    pos0 = j * BLOCK_ROWS * COLS + row * COLS + col

    def step(i, carry):
        m_vec, idx_vec = carry
        pos = pos0 + i * (STRIP * COLS)
        w = w_ref[pl.ds(i * STRIP, STRIP), :]
        bits = _threefry_bits(pos.astype(jnp.uint32))
        fbits = (bits >> jnp.uint32(9)) | jnp.uint32(0x3F800000)
        f = jax.lax.bitcast_convert_type(fbits, jnp.float32) - jnp.float32(1.0)
        u = jnp.maximum(f, jnp.float32(1.1754943508222875e-38))
        z = -jnp.log(-jnp.log(u)) + jnp.log(w)
        upd = z > m_vec
        m_vec = jnp.where(upd, z, m_vec)
        idx_vec = jnp.where(upd, pos, idx_vec)
        return m_vec, idx_vec

    m0 = jnp.where(j == 0, jnp.full((STRIP, COLS), -jnp.inf, jnp.float32),
                   m_acc[...])
    i0 = jnp.where(j == 0, jnp.full((STRIP, COLS), 2**31 - 1, jnp.int32),
                   idx_acc[...])
    m_vec, idx_vec = jax.lax.fori_loop(
        0, BLOCK_ROWS // STRIP, step, (m0, i0), unroll=8)
    m_acc[...] = m_vec
    idx_acc[...] = idx_vec

    @pl.when(j == GRID - 1)
    def _():
        m = jnp.max(m_vec)
        out_ref[0] = jnp.min(
            jnp.where(m_vec == m, idx_vec, jnp.int32(2**31 - 1)))


def kernel(weights):
    wp = jnp.pad(weights, (0, PAD - N)).reshape(ROWS, COLS)
    idx = pl.pallas_call(
        _body,
        grid=(GRID,),
        in_specs=[pl.BlockSpec((BLOCK_ROWS, COLS), lambda j: (j, 0))],
        out_specs=pl.BlockSpec(memory_space=pltpu.SMEM),
        out_shape=jax.ShapeDtypeStruct((1,), jnp.int32),
        scratch_shapes=[
            pltpu.VMEM((STRIP, COLS), jnp.float32),
            pltpu.VMEM((STRIP, COLS), jnp.int32),
        ],
    )(wp)
    return idx
